# baseline (speedup 1.0000x reference)
import functools

import jax
import jax.numpy as jnp
from jax import lax
from jax.experimental import pallas as pl
from jax.experimental.pallas import tpu as pltpu
from jax.experimental.pallas import tpu_sc as plsc

NND = 10000
DF = 128
NE = 320000
DE = 4
NC = 2
NS = 16
NW = NC * NS
NPAD = 10240
RPT = NPAD // NS
EPAD = NW * 80 * 128
EROWS = EPAD // 128
CH = 32
NBUF = 4
E64 = EPAD // CH
ERT = E64 // NW
TROWS = NPAD // 128
GRP = 16
NLAYERS = 5
_F32 = jnp.float32


def _es_body(eft_ref, we_ref, es_ref, mx_ref):
    i = pl.program_id(0)

    @pl.when(i == 0)
    def _init():
        mx_ref[...] = jnp.full((8, 128), -jnp.inf, _F32)

    eft = eft_ref[...]
    w = we_ref[...]
    for l in range(NLAYERS):
        es = eft[0] * w[l, 0]
        for k in range(1, DE):
            es = es + eft[k] * w[l, k]
        es_ref[l] = es
        mx_ref[l] = jnp.maximum(mx_ref[l], jnp.max(es))


def _es_call(eft, we_all):
    blk = 16
    return pl.pallas_call(
        _es_body,
        grid=(EROWS // blk,),
        in_specs=[
            pl.BlockSpec((DE, blk, 128), lambda i: (0, i, 0)),
            pl.BlockSpec((8, 128), lambda i: (0, 0)),
        ],
        out_specs=[
            pl.BlockSpec((NLAYERS, blk, 128), lambda i: (0, i, 0)),
            pl.BlockSpec((8, 128), lambda i: (0, 0)),
        ],
        out_shape=[
            jax.ShapeDtypeStruct((NLAYERS, EROWS, 128), _F32),
            jax.ShapeDtypeStruct((8, 128), _F32),
        ],
    )(eft, we_all)


def _dense_tail(h, W_ref, b_ref, asrc_ref, adst_ref, h2_ref, ss_ref, sd_ref):
    h2 = jnp.dot(h, W_ref[...], preferred_element_type=_F32) + b_ref[...]
    h2_ref[...] = h2
    ss_ref[...] = jnp.dot(h2, asrc_ref[...], preferred_element_type=_F32).reshape(TROWS, 128)
    sd_ref[...] = jnp.dot(h2, adst_ref[...], preferred_element_type=_F32).reshape(TROWS, 128)


def _dense_body(h_ref, W_ref, b_ref, asrc_ref, adst_ref, h2_ref, ss_ref, sd_ref):
    _dense_tail(h_ref[...], W_ref, b_ref, asrc_ref, adst_ref, h2_ref, ss_ref, sd_ref)


def _comb_dense_body(p_ref, den_ref, W_ref, b_ref, asrc_ref, adst_ref,
                     h2_ref, ss_ref, sd_ref):
    den = den_ref[0] + den_ref[1]
    h = jnp.maximum((p_ref[0] + p_ref[1]) / (den + 1e-9)[:, None], 0.0)
    _dense_tail(h, W_ref, b_ref, asrc_ref, adst_ref, h2_ref, ss_ref, sd_ref)


_DENSE_OUT = [
    jax.ShapeDtypeStruct((NPAD, DF), _F32),
    jax.ShapeDtypeStruct((TROWS, 128), _F32),
    jax.ShapeDtypeStruct((TROWS, 128), _F32),
]


def _dense_call(h, p):
    return pl.pallas_call(_dense_body, out_shape=_DENSE_OUT)(
        h, p["W"], p["b"], p["a_src"], p["a_dst"])


def _comb_dense_call(outp, denp, p):
    return pl.pallas_call(_comb_dense_body, out_shape=_DENSE_OUT)(
        outp, denp, p["W"], p["b"], p["a_src"], p["a_dst"])


def _tbl_max(tbl_ref):
    def body(i, acc):
        for k in range(8):
            acc = jnp.maximum(acc, tbl_ref[i, pl.ds(k * 16, 16)])
        return acc
    return lax.fori_loop(0, TROWS, body, jnp.full((16,), -jnp.inf, _F32))


def _lane_max(v, scratch_ref):
    scratch_ref[pl.ds(0, 16)] = v
    m = plsc.load_gather(scratch_ref, [jnp.zeros((16,), jnp.int32)])
    for i in range(1, 16):
        m = jnp.maximum(
            m, plsc.load_gather(scratch_ref, [jnp.full((16,), i, jnp.int32)]))
    return m


def _edge_kernel_body(h2, ssrc, sdst, es2d, src2d, dst2d, esm, zrows, zden,
                      outp, denp,
                      ssrc_v, sdst_v, srcbuf, dstbuf, esbuf, rows_a, rows_b,
                      rows_c, rows_d, exbuf, esmv, acc_sh, den_sh,
                      sem_a, sem_b, sem_c, sem_d):
    cid = lax.axis_index("c")
    sid = lax.axis_index("s")
    wid = cid * NS + sid

    pltpu.sync_copy(ssrc, ssrc_v)
    pltpu.sync_copy(sdst, sdst_v)
    pltpu.sync_copy(esm, esmv)

    c_bound = (_lane_max(_tbl_max(ssrc_v), exbuf)
               + _lane_max(_tbl_max(sdst_v), exbuf)
               + _lane_max(esmv[...], exbuf) - 20.0)

    pltpu.sync_copy(zrows, acc_sh.at[pl.ds(sid * RPT, RPT)])
    pltpu.sync_copy(zden, den_sh.at[pl.ds(sid * RPT, RPT)])
    plsc.subcore_barrier()

    def _process(c, rbuf, sem):
        for k in range(CH // 16):
            s16 = srcbuf[c, pl.ds(k * 16, 16)]
            d16 = dstbuf[c, pl.ds(k * 16, 16)]
            g1 = plsc.load_gather(ssrc_v, [s16 >> 7, s16 & 127])
            g2 = plsc.load_gather(sdst_v, [d16 >> 7, d16 & 127])
            sc = g1 + g2 + esbuf[c, pl.ds(k * 16, 16)]
            sc = jnp.where(sc >= 0, sc, 0.2 * sc)
            exbuf[pl.ds(k * 16, 16)] = jnp.exp(sc - c_bound)

        pltpu.make_async_copy(h2.at[pl.ds(0, CH)], rbuf, sem).wait()

        def scale(e, _):
            b = plsc.load_gather(exbuf, [jnp.full((16,), e, jnp.int32)])
            for k in range(8):
                rbuf[e, pl.ds(k * 16, 16)] = rbuf[e, pl.ds(k * 16, 16)] * b
            return 0
        lax.fori_loop(0, CH, scale, 0)

        pltpu.sync_copy(exbuf, den_sh.at[dstbuf.at[c]], add=True)
        pltpu.sync_copy(rbuf, acc_sh.at[dstbuf.at[c]], add=True)

    def group(g, _):
        base = wid * ERT + g * GRP
        pltpu.sync_copy(src2d.at[pl.ds(base, GRP)], srcbuf)
        pltpu.sync_copy(dst2d.at[pl.ds(base, GRP)], dstbuf)
        pltpu.sync_copy(es2d.at[pl.ds(base, GRP)], esbuf)

        ring = ((rows_a, sem_a), (rows_b, sem_b), (rows_c, sem_c),
                (rows_d, sem_d))
        for b in range(NBUF - 1):
            pltpu.async_copy(h2.at[srcbuf.at[b]], ring[b][0], ring[b][1])

        def quad(q, _):
            c0 = NBUF * q
            for b in range(NBUF):
                c = c0 + b
                nxt = c + NBUF - 1
                nb = (b + NBUF - 1) % NBUF

                @pl.when(nxt < GRP)
                def _():
                    pltpu.async_copy(h2.at[srcbuf.at[nxt]], ring[nb][0],
                                     ring[nb][1])
                _process(c, ring[b][0], ring[b][1])
            return 0

        lax.fori_loop(0, GRP // NBUF, quad, 0)
        return 0

    lax.fori_loop(0, ERT // GRP, group, 0)
    plsc.subcore_barrier()

    pltpu.sync_copy(acc_sh.at[pl.ds(sid * RPT, RPT)],
                    outp.at[cid, pl.ds(sid * RPT, RPT)])
    pltpu.sync_copy(den_sh.at[pl.ds(sid * RPT, RPT)],
                    denp.at[cid, pl.ds(sid * RPT, RPT)])


def _edge_call(h2, ss, sd, es2d, esm, src2d, dst2d, zrows, zden):
    mesh = plsc.VectorSubcoreMesh(core_axis_name="c", subcore_axis_name="s")
    f = pl.kernel(
        _edge_kernel_body,
        out_type=[
            jax.ShapeDtypeStruct((NC, NPAD, 128), _F32),
            jax.ShapeDtypeStruct((NC, NPAD), _F32),
        ],
        mesh=mesh,
        compiler_params=pltpu.CompilerParams(needs_layout_passes=False),
        scratch_types=[
            pltpu.VMEM((TROWS, 128), _F32),
            pltpu.VMEM((TROWS, 128), _F32),
            pltpu.VMEM((GRP, CH), jnp.int32),
            pltpu.VMEM((GRP, CH), jnp.int32),
            pltpu.VMEM((GRP, CH), _F32),
            pltpu.VMEM((CH, 128), _F32),
            pltpu.VMEM((CH, 128), _F32),
            pltpu.VMEM((CH, 128), _F32),
            pltpu.VMEM((CH, 128), _F32),
            pltpu.VMEM((CH,), _F32),
            pltpu.VMEM((16,), _F32),
            pltpu.VMEM_SHARED((NPAD, 128), _F32),
            pltpu.VMEM_SHARED((NPAD,), _F32),
            pltpu.SemaphoreType.DMA,
            pltpu.SemaphoreType.DMA,
            pltpu.SemaphoreType.DMA,
            pltpu.SemaphoreType.DMA,
        ],
    )
    return f(h2, ss, sd, es2d, src2d, dst2d, esm, zrows, zden)


def _hl_head_body(p_ref, den_ref, W1_ref, b1_ref, W2_ref, b2_ref, gW_ref,
                  gb_ref, high_ref, goal_ref):
    den = den_ref[0] + den_ref[1]
    h = jnp.maximum((p_ref[0] + p_ref[1]) / (den + 1e-9)[:, None], 0.0)
    rid = lax.broadcasted_iota(jnp.int32, (NPAD, DF), 0)
    hm = jnp.where(rid < NND, h, 0.0)
    gemb = jnp.sum(hm, axis=0, keepdims=True) / float(NND)
    hid = jnp.maximum(jnp.dot(gemb, W1_ref[...], preferred_element_type=_F32)
                      + b1_ref[...], 0.0)
    high = jnp.dot(hid, W2_ref[...], preferred_element_type=_F32) + b2_ref[...]
    goal = jnp.dot(high, gW_ref[...], preferred_element_type=_F32) + gb_ref[...]
    high_ref[...] = jnp.broadcast_to(high, (8, 128))
    goal_ref[...] = jnp.broadcast_to(goal, (8, 128))


def _ll_head_body(p_ref, den_ref, goal_ref, W1_ref, b1_ref, W2_ref, b2_ref,
                  low_ref):
    den = den_ref[0] + den_ref[1]
    h = jnp.maximum((p_ref[0] + p_ref[1]) / (den + 1e-9)[:, None], 0.0)
    cond = h + goal_ref[...][0]
    hid = jnp.maximum(jnp.dot(cond, W1_ref[...], preferred_element_type=_F32)
                      + b1_ref[...], 0.0)
    low_ref[...] = jnp.dot(hid, W2_ref[...], preferred_element_type=_F32) + b2_ref[...]


def kernel(node_features, edge_index, edge_features, hl_params, ll_params,
           hlp, llp, gp):
    src, dst, ef0, ef1, ef2, ef3 = lax.sort(
        (edge_index[0].astype(jnp.int32), edge_index[1].astype(jnp.int32),
         edge_features[:, 0], edge_features[:, 1], edge_features[:, 2],
         edge_features[:, 3]),
        num_keys=1)
    srcp = jnp.concatenate(
        [src, jnp.zeros((EPAD - NE,), jnp.int32)]).reshape(E64, CH)
    dstp = jnp.concatenate(
        [dst, jnp.full((EPAD - NE,), NPAD - 1, jnp.int32)]).reshape(E64, CH)
    eft = jnp.stack([jnp.pad(c, (0, EPAD - NE)) for c in (ef0, ef1, ef2, ef3)]
                    ).reshape(DE, EROWS, 128)

    layers = list(hl_params) + list(ll_params)
    we_all = jnp.zeros((8, 128), _F32)
    for l, p in enumerate(layers):
        we_all = we_all.at[l, :DE].set(p["We"])
    es_all, esmax = _es_call(eft, we_all)
    es_all = es_all.reshape(NLAYERS, E64, CH)

    xpad = jnp.pad(node_features, ((0, NPAD - NND), (0, 0)))
    zrows = jnp.zeros((RPT, 128), _F32)
    zden = jnp.zeros((RPT,), _F32)

    def run_encoder(params, es_off):
        outp = denp = None
        for i, p in enumerate(params):
            if i == 0:
                h2, ss, sd = _dense_call(xpad, p)
            else:
                h2, ss, sd = _comb_dense_call(outp, denp, p)
            outp, denp = _edge_call(h2, ss, sd, es_all[es_off + i],
                                    esmax[es_off + i, :16], srcp, dstp,
                                    zrows, zden)
        return outp, denp

    hp, hd = run_encoder(hl_params, 0)

    hW1, hb1, hW2, hb2 = hlp
    gW, gb = gp
    high_buf, goal_buf = pl.pallas_call(
        _hl_head_body,
        out_shape=[jax.ShapeDtypeStruct((8, 128), _F32),
                   jax.ShapeDtypeStruct((8, 128), _F32)],
    )(hp, hd, hW1, hb1,
      jnp.pad(hW2, ((0, 0), (0, 128 - hW2.shape[1]))),
      jnp.pad(hb2, (0, 128 - hb2.shape[0])),
      jnp.pad(gW, ((0, 128 - gW.shape[0]), (0, 0))), gb)

    lp, ld = run_encoder(ll_params, 2)

    lW1, lb1, lW2, lb2 = llp
    low_pad = pl.pallas_call(
        _ll_head_body,
        out_shape=jax.ShapeDtypeStruct((NPAD, 128), _F32),
    )(lp, ld, goal_buf, lW1, lb1,
      jnp.pad(lW2, ((0, 0), (0, 128 - lW2.shape[1]))),
      jnp.pad(lb2, (0, 128 - lb2.shape[0])))

    high_actions = high_buf[0, :hb2.shape[0]]
    low_actions = low_pad[:NND, :lb2.shape[0]]
    return (high_actions, low_actions)


# speedup vs baseline: 1.3257x; 1.3257x over previous
import functools

import jax
import jax.numpy as jnp
from jax import lax
from jax.experimental import pallas as pl
from jax.experimental.pallas import tpu as pltpu
from jax.experimental.pallas import tpu_sc as plsc

NND = 10000
DF = 128
NE = 320000
DE = 4
NC = 2
NS = 16
NW = NC * NS
NPAD = 10240
RPT = NPAD // NS
EPAD = NW * 80 * 128
EROWS = EPAD // 128
CH = 32
NBUF = 4
E64 = EPAD // CH
ERT = E64 // NW
TROWS = NPAD // 128
GRP = 16
NLAYERS = 5
_F32 = jnp.float32


def _es_body(eft_ref, we_ref, es_ref, mx_ref):
    i = pl.program_id(0)

    @pl.when(i == 0)
    def _init():
        mx_ref[...] = jnp.full((8, 128), -jnp.inf, _F32)

    eft = eft_ref[...]
    w = we_ref[...]
    for l in range(NLAYERS):
        es = eft[0] * w[l, 0]
        for k in range(1, DE):
            es = es + eft[k] * w[l, k]
        es_ref[l] = es
        mx_ref[l] = jnp.maximum(mx_ref[l], jnp.max(es))


def _es_call(eft, we_all):
    blk = 16
    return pl.pallas_call(
        _es_body,
        grid=(EROWS // blk,),
        in_specs=[
            pl.BlockSpec((DE, blk, 128), lambda i: (0, i, 0)),
            pl.BlockSpec((8, 128), lambda i: (0, 0)),
        ],
        out_specs=[
            pl.BlockSpec((NLAYERS, blk, 128), lambda i: (0, i, 0)),
            pl.BlockSpec((8, 128), lambda i: (0, 0)),
        ],
        out_shape=[
            jax.ShapeDtypeStruct((NLAYERS, EROWS, 128), _F32),
            jax.ShapeDtypeStruct((8, 128), _F32),
        ],
    )(eft, we_all)


def _dense_tail(h, W_ref, b_ref, asrc_ref, adst_ref, h2_ref, ss_ref, sd_ref):
    h2 = jnp.dot(h, W_ref[...], preferred_element_type=_F32) + b_ref[...]
    h2_ref[...] = h2
    ss_ref[...] = jnp.dot(h2, asrc_ref[...], preferred_element_type=_F32).reshape(TROWS, 128)
    sd_ref[...] = jnp.dot(h2, adst_ref[...], preferred_element_type=_F32).reshape(TROWS, 128)


def _dense_body(h_ref, W_ref, b_ref, asrc_ref, adst_ref, h2_ref, ss_ref, sd_ref):
    _dense_tail(h_ref[...], W_ref, b_ref, asrc_ref, adst_ref, h2_ref, ss_ref, sd_ref)


def _comb_dense_body(p_ref, den_ref, W_ref, b_ref, asrc_ref, adst_ref,
                     h2_ref, ss_ref, sd_ref):
    den = den_ref[0] + den_ref[1]
    h = jnp.maximum((p_ref[0] + p_ref[1]) / (den + 1e-9)[:, None], 0.0)
    _dense_tail(h, W_ref, b_ref, asrc_ref, adst_ref, h2_ref, ss_ref, sd_ref)


_DENSE_OUT = [
    jax.ShapeDtypeStruct((NPAD, DF), _F32),
    jax.ShapeDtypeStruct((TROWS, 128), _F32),
    jax.ShapeDtypeStruct((TROWS, 128), _F32),
]


def _dense_call(h, p):
    return pl.pallas_call(_dense_body, out_shape=_DENSE_OUT)(
        h, p["W"], p["b"], p["a_src"], p["a_dst"])


def _comb_dense_call(outp, denp, p):
    return pl.pallas_call(_comb_dense_body, out_shape=_DENSE_OUT)(
        outp, denp, p["W"], p["b"], p["a_src"], p["a_dst"])


def _tbl_max(tbl_ref):
    def body(i, acc):
        for k in range(8):
            acc = jnp.maximum(acc, tbl_ref[i, pl.ds(k * 16, 16)])
        return acc
    return lax.fori_loop(0, TROWS, body, jnp.full((16,), -jnp.inf, _F32))


def _lane_max(v, scratch_ref):
    scratch_ref[pl.ds(0, 16)] = v
    m = plsc.load_gather(scratch_ref, [jnp.zeros((16,), jnp.int32)])
    for i in range(1, 16):
        m = jnp.maximum(
            m, plsc.load_gather(scratch_ref, [jnp.full((16,), i, jnp.int32)]))
    return m


def _edge_kernel_body(h2, ssrc, sdst, es2d, src2d, dst2d, esm, zrows, zden,
                      outp, denp,
                      ssrc_v, sdst_v, srcbuf, dstbuf, esbuf, rows_a, rows_b,
                      rows_c, rows_d, exbuf, esmv, acc_sh, den_sh,
                      sem_a, sem_b, sem_c, sem_d):
    cid = lax.axis_index("c")
    sid = lax.axis_index("s")
    wid = cid * NS + sid

    pltpu.sync_copy(ssrc, ssrc_v)
    pltpu.sync_copy(sdst, sdst_v)
    pltpu.sync_copy(esm, esmv)

    c_bound = (_lane_max(_tbl_max(ssrc_v), exbuf)
               + _lane_max(_tbl_max(sdst_v), exbuf)
               + _lane_max(esmv[...], exbuf) - 20.0)

    pltpu.sync_copy(zrows, acc_sh.at[pl.ds(sid * RPT, RPT)])
    pltpu.sync_copy(zden, den_sh.at[pl.ds(sid * RPT, RPT)])
    plsc.subcore_barrier()

    def _process(c, rbuf, sem):
        for k in range(CH // 16):
            s16 = srcbuf[c, pl.ds(k * 16, 16)]
            d16 = dstbuf[c, pl.ds(k * 16, 16)]
            g1 = plsc.load_gather(ssrc_v, [s16 >> 7, s16 & 127])
            g2 = plsc.load_gather(sdst_v, [d16 >> 7, d16 & 127])
            sc = g1 + g2 + esbuf[c, pl.ds(k * 16, 16)]
            sc = jnp.where(sc >= 0, sc, 0.2 * sc)
            exbuf[pl.ds(k * 16, 16)] = jnp.exp(sc - c_bound)

        pltpu.make_async_copy(h2.at[pl.ds(0, CH)], rbuf, sem).wait()

        def scale(e, _):
            b = plsc.load_gather(exbuf, [jnp.full((16,), e, jnp.int32)])
            for k in range(8):
                rbuf[e, pl.ds(k * 16, 16)] = rbuf[e, pl.ds(k * 16, 16)] * b
            return 0
        lax.fori_loop(0, CH, scale, 0)

        pltpu.sync_copy(exbuf, den_sh.at[dstbuf.at[c]], add=True)
        pltpu.sync_copy(rbuf, acc_sh.at[dstbuf.at[c]], add=True)

    def group(g, _):
        base = wid * ERT + g * GRP
        pltpu.sync_copy(src2d.at[pl.ds(base, GRP)], srcbuf)
        pltpu.sync_copy(dst2d.at[pl.ds(base, GRP)], dstbuf)
        pltpu.sync_copy(es2d.at[pl.ds(base, GRP)], esbuf)

        ring = ((rows_a, sem_a), (rows_b, sem_b), (rows_c, sem_c),
                (rows_d, sem_d))
        for b in range(NBUF - 1):
            pltpu.async_copy(h2.at[srcbuf.at[b]], ring[b][0], ring[b][1])

        def quad(q, _):
            c0 = NBUF * q
            for b in range(NBUF):
                c = c0 + b
                nxt = c + NBUF - 1
                nb = (b + NBUF - 1) % NBUF

                @pl.when(nxt < GRP)
                def _():
                    pltpu.async_copy(h2.at[srcbuf.at[nxt]], ring[nb][0],
                                     ring[nb][1])
                _process(c, ring[b][0], ring[b][1])
            return 0

        lax.fori_loop(0, GRP // NBUF, quad, 0)
        return 0

    lax.fori_loop(0, ERT // GRP, group, 0)
    plsc.subcore_barrier()

    pltpu.sync_copy(acc_sh.at[pl.ds(sid * RPT, RPT)],
                    outp.at[cid, pl.ds(sid * RPT, RPT)])
    pltpu.sync_copy(den_sh.at[pl.ds(sid * RPT, RPT)],
                    denp.at[cid, pl.ds(sid * RPT, RPT)])


def _edge_call(h2, ss, sd, es2d, esm, src2d, dst2d, zrows, zden):
    mesh = plsc.VectorSubcoreMesh(core_axis_name="c", subcore_axis_name="s")
    f = pl.kernel(
        _edge_kernel_body,
        out_type=[
            jax.ShapeDtypeStruct((NC, NPAD, 128), _F32),
            jax.ShapeDtypeStruct((NC, NPAD), _F32),
        ],
        mesh=mesh,
        compiler_params=pltpu.CompilerParams(needs_layout_passes=False),
        scratch_types=[
            pltpu.VMEM((TROWS, 128), _F32),
            pltpu.VMEM((TROWS, 128), _F32),
            pltpu.VMEM((GRP, CH), jnp.int32),
            pltpu.VMEM((GRP, CH), jnp.int32),
            pltpu.VMEM((GRP, CH), _F32),
            pltpu.VMEM((CH, 128), _F32),
            pltpu.VMEM((CH, 128), _F32),
            pltpu.VMEM((CH, 128), _F32),
            pltpu.VMEM((CH, 128), _F32),
            pltpu.VMEM((CH,), _F32),
            pltpu.VMEM((16,), _F32),
            pltpu.VMEM_SHARED((NPAD, 128), _F32),
            pltpu.VMEM_SHARED((NPAD,), _F32),
            pltpu.SemaphoreType.DMA,
            pltpu.SemaphoreType.DMA,
            pltpu.SemaphoreType.DMA,
            pltpu.SemaphoreType.DMA,
        ],
    )
    return f(h2, ss, sd, es2d, src2d, dst2d, esm, zrows, zden)


def _hl_head_body(p_ref, den_ref, W1_ref, b1_ref, W2_ref, b2_ref, gW_ref,
                  gb_ref, high_ref, goal_ref):
    den = den_ref[0] + den_ref[1]
    h = jnp.maximum((p_ref[0] + p_ref[1]) / (den + 1e-9)[:, None], 0.0)
    rid = lax.broadcasted_iota(jnp.int32, (NPAD, DF), 0)
    hm = jnp.where(rid < NND, h, 0.0)
    gemb = jnp.sum(hm, axis=0, keepdims=True) / float(NND)
    hid = jnp.maximum(jnp.dot(gemb, W1_ref[...], preferred_element_type=_F32)
                      + b1_ref[...], 0.0)
    high = jnp.dot(hid, W2_ref[...], preferred_element_type=_F32) + b2_ref[...]
    goal = jnp.dot(high, gW_ref[...], preferred_element_type=_F32) + gb_ref[...]
    high_ref[...] = jnp.broadcast_to(high, (8, 128))
    goal_ref[...] = jnp.broadcast_to(goal, (8, 128))


def _ll_head_body(p_ref, den_ref, goal_ref, W1_ref, b1_ref, W2_ref, b2_ref,
                  low_ref):
    den = den_ref[0] + den_ref[1]
    h = jnp.maximum((p_ref[0] + p_ref[1]) / (den + 1e-9)[:, None], 0.0)
    cond = h + goal_ref[...][0]
    hid = jnp.maximum(jnp.dot(cond, W1_ref[...], preferred_element_type=_F32)
                      + b1_ref[...], 0.0)
    low_ref[...] = jnp.dot(hid, W2_ref[...], preferred_element_type=_F32) + b2_ref[...]


def kernel(node_features, edge_index, edge_features, hl_params, ll_params,
           hlp, llp, gp):
    src = edge_index[0].astype(jnp.int32)
    dst = edge_index[1].astype(jnp.int32)
    srcp = jnp.concatenate(
        [src, jnp.zeros((EPAD - NE,), jnp.int32)]).reshape(E64, CH)
    dstp = jnp.concatenate(
        [dst, jnp.full((EPAD - NE,), NPAD - 1, jnp.int32)]).reshape(E64, CH)
    eft = jnp.pad(edge_features, ((0, EPAD - NE), (0, 0))).T.reshape(DE, EROWS, 128)

    layers = list(hl_params) + list(ll_params)
    we_all = jnp.zeros((8, 128), _F32)
    for l, p in enumerate(layers):
        we_all = we_all.at[l, :DE].set(p["We"])
    es_all, esmax = _es_call(eft, we_all)
    es_all = es_all.reshape(NLAYERS, E64, CH)

    xpad = jnp.pad(node_features, ((0, NPAD - NND), (0, 0)))
    zrows = jnp.zeros((RPT, 128), _F32)
    zden = jnp.zeros((RPT,), _F32)

    def run_encoder(params, es_off):
        outp = denp = None
        for i, p in enumerate(params):
            if i == 0:
                h2, ss, sd = _dense_call(xpad, p)
            else:
                h2, ss, sd = _comb_dense_call(outp, denp, p)
            outp, denp = _edge_call(h2, ss, sd, es_all[es_off + i],
                                    esmax[es_off + i, :16], srcp, dstp,
                                    zrows, zden)
        return outp, denp

    hp, hd = run_encoder(hl_params, 0)

    hW1, hb1, hW2, hb2 = hlp
    gW, gb = gp
    high_buf, goal_buf = pl.pallas_call(
        _hl_head_body,
        out_shape=[jax.ShapeDtypeStruct((8, 128), _F32),
                   jax.ShapeDtypeStruct((8, 128), _F32)],
    )(hp, hd, hW1, hb1,
      jnp.pad(hW2, ((0, 0), (0, 128 - hW2.shape[1]))),
      jnp.pad(hb2, (0, 128 - hb2.shape[0])),
      jnp.pad(gW, ((0, 128 - gW.shape[0]), (0, 0))), gb)

    lp, ld = run_encoder(ll_params, 2)

    lW1, lb1, lW2, lb2 = llp
    low_pad = pl.pallas_call(
        _ll_head_body,
        out_shape=jax.ShapeDtypeStruct((NPAD, 128), _F32),
    )(lp, ld, goal_buf, lW1, lb1,
      jnp.pad(lW2, ((0, 0), (0, 128 - lW2.shape[1]))),
      jnp.pad(lb2, (0, 128 - lb2.shape[0])))

    high_actions = high_buf[0, :hb2.shape[0]]
    low_actions = low_pad[:NND, :lb2.shape[0]]
    return (high_actions, low_actions)
